# Initial kernel scaffold; baseline (speedup 1.0000x reference)
import jax
import jax.numpy as jnp
from jax.experimental import pallas as pl


def kernel(feature, s_edge_index, t_edge_index, poi_edge_index, W_s, a_src_s, a_dst_s, W_t, a_src_t, a_dst_t, W_p, a_src_p, a_dst_p, Wq, Wk, mv_w, mv_b):
    raise NotImplementedError("write your pallas kernel here")


# speedup vs baseline: 58.6805x; 58.6805x over previous
import functools

import jax
import jax.numpy as jnp
from jax import lax
from jax.experimental import pallas as pl
from jax.experimental.pallas import tpu as pltpu
from jax.experimental.pallas import tpu_sc as plsc

N = 1024
D_IN = 256
D_OUT = 64
H = 12
E = 32768
HID = 48
ALPHA = 0.8
BETA = 0.5

NC = 2
NS = 16
NW = NC * NS
ROWS_PER_W = N // NW
LANES = 16


def _build_counts(edges_flat):
    mesh = plsc.VectorSubcoreMesh(core_axis_name="c", subcore_axis_name="s")

    @functools.partial(
        pl.kernel,
        out_type=jax.ShapeDtypeStruct((3 * N * N,), jnp.float32),
        mesh=mesh,
        scratch_types=[
            pltpu.VMEM((E,), jnp.int32),
            pltpu.VMEM((E,), jnp.int32),
            pltpu.VMEM((ROWS_PER_W * N,), jnp.float32),
        ],
        compiler_params=pltpu.CompilerParams(needs_layout_passes=False),
    )
    def sc_kernel(edges_hbm, out_hbm, src_v, dst_v, c_v):
        wid = lax.axis_index("s") * NC + lax.axis_index("c")
        lo = wid * ROWS_PER_W
        ones = jnp.full((LANES,), 1.0, jnp.float32)
        zeros = jnp.zeros((LANES,), jnp.float32)

        for v in range(3):
            @pl.loop(0, ROWS_PER_W * N, step=LANES)
            def _(i):
                c_v[pl.ds(i, LANES)] = zeros

            pltpu.sync_copy(edges_hbm.at[pl.ds(v * 2 * E, E)], src_v)
            pltpu.sync_copy(edges_hbm.at[pl.ds(v * 2 * E + E, E)], dst_v)

            @pl.loop(0, E, step=LANES)
            def _(j):
                s16 = src_v[pl.ds(j, LANES)]
                d16 = dst_v[pl.ds(j, LANES)]
                rel = d16 - lo
                msk = (rel >= 0) & (rel < ROWS_PER_W)
                li = jnp.where(msk, rel * N + s16, 0)
                plsc.addupdate_scatter(c_v, [li], ones, mask=msk)

            pltpu.sync_copy(
                c_v, out_hbm.at[pl.ds(v * N * N + lo * N, ROWS_PER_W * N)])

    return sc_kernel(edges_flat)


def _gat_body(f_ref, w_ref, asrc_ref, adst_ref, c_ref, o_ref):
    h_idx = pl.program_id(1)
    hb = jnp.dot(f_ref[...], w_ref[0], preferred_element_type=jnp.float32)
    a_s = asrc_ref[0]
    a_d = adst_ref[0]
    e_src = lax.dot_general(a_s, hb, (((1,), (1,)), ((), ())),
                            preferred_element_type=jnp.float32)
    e_dst = jnp.dot(hb, a_d.reshape(D_OUT, 1),
                    preferred_element_type=jnp.float32)
    e = e_dst + e_src
    e = jnp.where(e >= 0, e, 0.2 * e)
    c = c_ref[0]
    msk = c > 0
    em = jnp.where(msk, e, jnp.float32(-1e30))
    m = jnp.max(em, axis=1, keepdims=True)
    m = jnp.where(m > jnp.float32(-1e29), m, 0.0)
    ex = jnp.where(msk, jnp.exp(e - m), 0.0)
    cex = c * ex
    denom = jnp.sum(cex, axis=1, keepdims=True)
    wmat = cex / (denom + 1e-9)
    out_h = jnp.dot(wmat, hb, preferred_element_type=jnp.float32)
    contrib = jnp.maximum(out_h, 0.0) * (1.0 / H)

    @pl.when(h_idx == 0)
    def _():
        o_ref[0] = contrib

    @pl.when(h_idx != 0)
    def _():
        o_ref[0] += contrib


def _gat_dense(feature, w_all, a_src_all, a_dst_all, counts):
    return pl.pallas_call(
        _gat_body,
        grid=(3, H),
        in_specs=[
            pl.BlockSpec((N, D_IN), lambda v, h: (0, 0)),
            pl.BlockSpec((1, D_IN, D_OUT), lambda v, h: (v * H + h, 0, 0)),
            pl.BlockSpec((1, 1, D_OUT), lambda v, h: (v * H + h, 0, 0)),
            pl.BlockSpec((1, 1, D_OUT), lambda v, h: (v * H + h, 0, 0)),
            pl.BlockSpec((1, N, N), lambda v, h: (v, 0, 0)),
        ],
        out_specs=pl.BlockSpec((1, N, D_OUT), lambda v, h: (v, 0, 0)),
        out_shape=jax.ShapeDtypeStruct((3, N, D_OUT), jnp.float32),
    )(feature, w_all, a_src_all, a_dst_all, counts)


def _fuse_body(views_ref, wq_ref, wk_ref, mvw_ref, mvb_ref, o_ref):
    scale = float(1.0 / (HID / N) ** 0.5)
    mvw = mvw_ref[...]
    mixeds = []
    omegas = []
    for v in range(3):
        xv = views_ref[v]
        q = jnp.dot(xv, wq_ref[...], preferred_element_type=jnp.float32)
        k = jnp.dot(xv, wk_ref[...], preferred_element_type=jnp.float32)
        t = lax.dot_general(k, xv, (((0,), (0,)), ((), ())),
                            preferred_element_type=jnp.float32)
        fused = jnp.dot(q, t, preferred_element_type=jnp.float32) * scale
        mixed = ALPHA * fused + (1.0 - ALPHA) * xv
        mixeds.append(mixed)
        s = jnp.sum(mixed * mvw, axis=1, keepdims=True)
        s = jnp.sum(s, axis=0, keepdims=True)
        omegas.append(jax.nn.sigmoid(s + mvb_ref[...]))
    mv_out = (omegas[0] * mixeds[0] + omegas[1] * mixeds[1]
              + omegas[2] * mixeds[2])
    for v in range(3):
        o_ref[v] = BETA * mixeds[v] + (1.0 - BETA) * mv_out


def _fuse(views, wq, wk, mvw2d, mvb2d):
    return pl.pallas_call(
        _fuse_body,
        out_shape=jax.ShapeDtypeStruct((3, N, D_OUT), jnp.float32),
    )(views, wq, wk, mvw2d, mvb2d)


def kernel(feature, s_edge_index, t_edge_index, poi_edge_index,
           W_s, a_src_s, a_dst_s, W_t, a_src_t, a_dst_t,
           W_p, a_src_p, a_dst_p, Wq, Wk, mv_w, mv_b):
    edges_flat = jnp.concatenate([
        s_edge_index.astype(jnp.int32).reshape(-1),
        t_edge_index.astype(jnp.int32).reshape(-1),
        poi_edge_index.astype(jnp.int32).reshape(-1),
    ])
    counts = _build_counts(edges_flat).reshape(3, N, N)

    w_all = (jnp.stack([W_s, W_t, W_p])
             .reshape(3, D_IN, H, D_OUT)
             .transpose(0, 2, 1, 3)
             .reshape(3 * H, D_IN, D_OUT))
    a_src_all = jnp.stack([a_src_s, a_src_t, a_src_p]).reshape(3 * H, 1, D_OUT)
    a_dst_all = jnp.stack([a_dst_s, a_dst_t, a_dst_p]).reshape(3 * H, 1, D_OUT)

    views = _gat_dense(feature, w_all, a_src_all, a_dst_all, counts)

    mvw2d = mv_w.reshape(N, D_OUT)
    mvb2d = jnp.reshape(mv_b, (1, 1)).astype(jnp.float32)
    return _fuse(views, Wq, Wk, mvw2d, mvb2d)
